# Initial kernel scaffold; baseline (speedup 1.0000x reference)
import jax
import jax.numpy as jnp
from jax.experimental import pallas as pl


def kernel(input, weights):
    raise NotImplementedError("write your pallas kernel here")


# speedup vs baseline: 1.9068x; 1.9068x over previous
import functools

import jax
import jax.numpy as jnp
from jax import lax
from jax.experimental import pallas as pl
from jax.experimental.pallas import tpu as pltpu
from jax.experimental.pallas import tpu_sc as plsc

_PAD = 1
_NC, _NS = 2, 16
_NW = _NC * _NS
_CHUNK = 32


def _positions_body(inp_ref, out_ref):
    x = inp_ref[...]
    rows, seq = x.shape
    mask = (x != _PAD).astype(jnp.int32)
    s = mask
    k = 1
    while k < seq:
        shifted = jnp.concatenate(
            [jnp.zeros((rows, k), jnp.int32), s[:, : seq - k]], axis=1
        )
        s = s + shifted
        k *= 2
    out_ref[...] = s * mask + _PAD


@functools.lru_cache(maxsize=None)
def _build(bsz, seq, vocab, dim):
    positions = pl.pallas_call(
        _positions_body,
        out_shape=jax.ShapeDtypeStruct((bsz, seq), jnp.int32),
    )

    b_total = bsz * seq
    b_per_w = b_total // _NW
    n_chunks = b_per_w // _CHUNK
    assert b_per_w * _NW == b_total and n_chunks * _CHUNK == b_per_w

    mesh = plsc.VectorSubcoreMesh(
        core_axis_name="c", subcore_axis_name="s",
        num_cores=_NC, num_subcores=_NS,
    )

    @functools.partial(
        pl.kernel,
        out_type=jax.ShapeDtypeStruct((b_total, dim), jnp.float32),
        mesh=mesh,
        scratch_types=[
            pltpu.VMEM((b_per_w,), jnp.int32),
            pltpu.VMEM((_CHUNK, dim), jnp.float32),
            pltpu.VMEM((_CHUNK, dim), jnp.float32),
            pltpu.SemaphoreType.DMA,
            pltpu.SemaphoreType.DMA,
            pltpu.SemaphoreType.DMA,
            pltpu.SemaphoreType.DMA,
        ],
    )
    def gather(pos_hbm, table_hbm, out_hbm, idx_v, buf0, buf1, g0, g1, s0, s1):
        wid = lax.axis_index("s") * _NC + lax.axis_index("c")
        base = wid * b_per_w
        pltpu.sync_copy(pos_hbm.at[pl.ds(base, b_per_w)], idx_v)

        def step(i, _):
            off = pl.multiple_of(i * _CHUNK, _CHUNK)
            cp = pltpu.async_copy(
                table_hbm.at[idx_v.at[pl.ds(off, _CHUNK)]], buf0, g0
            )
            cp.wait()
            pltpu.sync_copy(buf0, out_hbm.at[pl.ds(base + off, _CHUNK)])
            return 0

        lax.fori_loop(0, n_chunks, step, 0)

    def run(inp, weights):
        pos = positions(inp)
        flat = gather(pos.reshape(b_total), weights)
        return flat.reshape(bsz, seq, dim)

    return run


@jax.jit
def kernel(input, weights):
    bsz, seq = input.shape
    vocab, dim = weights.shape
    run = _build(bsz, seq, vocab, dim)
    return run(input.astype(jnp.int32), weights.astype(jnp.float32))


# speedup vs baseline: 2.2258x; 1.1673x over previous
import functools

import jax
import jax.numpy as jnp
from jax import lax
from jax.experimental import pallas as pl
from jax.experimental.pallas import tpu as pltpu
from jax.experimental.pallas import tpu_sc as plsc

_PAD = 1
_NC, _NS = 2, 16
_NW = _NC * _NS
_CHUNK = 32


def _positions_body(inp_ref, out_ref):
    x = inp_ref[...]
    rows, seq = x.shape
    mask = (x != _PAD).astype(jnp.int32)
    s = mask
    k = 1
    while k < seq:
        shifted = jnp.concatenate(
            [jnp.zeros((rows, k), jnp.int32), s[:, : seq - k]], axis=1
        )
        s = s + shifted
        k *= 2
    out_ref[...] = s * mask + _PAD


@functools.lru_cache(maxsize=None)
def _build(bsz, seq, vocab, dim):
    positions = pl.pallas_call(
        _positions_body,
        out_shape=jax.ShapeDtypeStruct((bsz, seq), jnp.int32),
    )

    b_total = bsz * seq
    b_per_w = b_total // _NW
    n_chunks = b_per_w // _CHUNK
    assert b_per_w * _NW == b_total and n_chunks * _CHUNK == b_per_w

    mesh = plsc.VectorSubcoreMesh(
        core_axis_name="c", subcore_axis_name="s",
        num_cores=_NC, num_subcores=_NS,
    )

    @functools.partial(
        pl.kernel,
        out_type=jax.ShapeDtypeStruct((b_total, dim), jnp.float32),
        mesh=mesh,
        scratch_types=[
            pltpu.VMEM((b_per_w,), jnp.int32),
            pltpu.VMEM((_CHUNK, dim), jnp.float32),
            pltpu.VMEM((_CHUNK, dim), jnp.float32),
            pltpu.SemaphoreType.DMA,
            pltpu.SemaphoreType.DMA,
            pltpu.SemaphoreType.DMA,
            pltpu.SemaphoreType.DMA,
        ],
    )
    def gather(pos_hbm, table_hbm, out_hbm, idx_v, buf0, buf1, g0, g1, s0, s1):
        wid = lax.axis_index("s") * _NC + lax.axis_index("c")
        base = wid * b_per_w
        pltpu.sync_copy(pos_hbm.at[pl.ds(base, b_per_w)], idx_v)

        def start_g(i, buf, sem):
            off = pl.multiple_of(i * _CHUNK, _CHUNK)
            pltpu.async_copy(table_hbm.at[idx_v.at[pl.ds(off, _CHUNK)]], buf, sem)

        def start_s(i, buf, sem):
            off = pl.multiple_of(i * _CHUNK, _CHUNK)
            pltpu.async_copy(buf, out_hbm.at[pl.ds(base + off, _CHUNK)], sem)

        def wait_g(buf, sem):
            pltpu.make_async_copy(
                table_hbm.at[idx_v.at[pl.ds(0, _CHUNK)]], buf, sem
            ).wait()

        def wait_s(buf, sem):
            pltpu.make_async_copy(buf, out_hbm.at[pl.ds(base, _CHUNK)], sem).wait()

        n_pairs = n_chunks // 2
        start_g(0, buf0, g0)
        start_g(1, buf1, g1)

        def pair(p, _):
            i = 2 * p
            wait_g(buf0, g0)
            start_s(i, buf0, s0)
            wait_g(buf1, g1)
            start_s(i + 1, buf1, s1)
            wait_s(buf0, s0)

            @pl.when(p + 1 < n_pairs)
            def _():
                start_g(i + 2, buf0, g0)

            wait_s(buf1, s1)

            @pl.when(p + 1 < n_pairs)
            def _():
                start_g(i + 3, buf1, g1)

            return 0

        lax.fori_loop(0, n_pairs, pair, 0)

    def run(inp, weights):
        pos = positions(inp)
        flat = gather(pos.reshape(b_total), weights)
        return flat.reshape(bsz, seq, dim)

    return run


@jax.jit
def kernel(input, weights):
    bsz, seq = input.shape
    vocab, dim = weights.shape
    run = _build(bsz, seq, vocab, dim)
    return run(input.astype(jnp.int32), weights.astype(jnp.float32))


# speedup vs baseline: 2.2817x; 1.0251x over previous
import functools

import jax
import jax.numpy as jnp
from jax import lax
from jax.experimental import pallas as pl
from jax.experimental.pallas import tpu as pltpu
from jax.experimental.pallas import tpu_sc as plsc

_PAD = 1
_NC, _NS = 2, 16
_NW = _NC * _NS
_CHUNK = 16
_NBUF = 4


def _positions_body(inp_ref, out_ref):
    x = inp_ref[...]
    rows, seq = x.shape
    mask = (x != _PAD).astype(jnp.int32)
    s = mask
    k = 1
    while k < seq:
        shifted = jnp.concatenate(
            [jnp.zeros((rows, k), jnp.int32), s[:, : seq - k]], axis=1
        )
        s = s + shifted
        k *= 2
    out_ref[...] = s * mask + _PAD


@functools.lru_cache(maxsize=None)
def _build(bsz, seq, vocab, dim):
    positions = pl.pallas_call(
        _positions_body,
        out_shape=jax.ShapeDtypeStruct((bsz, seq), jnp.int32),
    )

    b_total = bsz * seq
    b_per_w = b_total // _NW
    n_chunks = b_per_w // _CHUNK
    assert b_per_w * _NW == b_total and n_chunks * _CHUNK == b_per_w

    mesh = plsc.VectorSubcoreMesh(
        core_axis_name="c", subcore_axis_name="s",
        num_cores=_NC, num_subcores=_NS,
    )

    @functools.partial(
        pl.kernel,
        out_type=jax.ShapeDtypeStruct((b_total, dim), jnp.float32),
        mesh=mesh,
        scratch_types=[
            pltpu.VMEM((b_per_w,), jnp.int32),
        ]
        + [pltpu.VMEM((_CHUNK, dim), jnp.float32) for _ in range(_NBUF)]
        + [pltpu.SemaphoreType.DMA for _ in range(2 * _NBUF)],
    )
    def gather(pos_hbm, table_hbm, out_hbm, idx_v, *rest):
        bufs = rest[:_NBUF]
        gsems = rest[_NBUF : 2 * _NBUF]
        ssems = rest[2 * _NBUF : 3 * _NBUF]
        wid = lax.axis_index("s") * _NC + lax.axis_index("c")
        base = wid * b_per_w
        pltpu.sync_copy(pos_hbm.at[pl.ds(base, b_per_w)], idx_v)

        def start_g(i, k):
            off = pl.multiple_of(i * _CHUNK, _CHUNK)
            pltpu.async_copy(
                table_hbm.at[idx_v.at[pl.ds(off, _CHUNK)]], bufs[k], gsems[k]
            )

        def start_s(i, k):
            off = pl.multiple_of(i * _CHUNK, _CHUNK)
            pltpu.async_copy(bufs[k], out_hbm.at[pl.ds(base + off, _CHUNK)], ssems[k])

        def wait_g(k):
            pltpu.make_async_copy(
                table_hbm.at[idx_v.at[pl.ds(0, _CHUNK)]], bufs[k], gsems[k]
            ).wait()

        def wait_s(k):
            pltpu.make_async_copy(
                bufs[k], out_hbm.at[pl.ds(base, _CHUNK)], ssems[k]
            ).wait()

        n_groups = n_chunks // _NBUF
        for k in range(_NBUF):
            start_g(k, k)

        def group(g, _):
            i0 = g * _NBUF
            for k in range(_NBUF):
                wait_g(k)
                start_s(i0 + k, k)
            for k in range(_NBUF):
                wait_s(k)

                @pl.when(g + 1 < n_groups)
                def _(k=k):
                    start_g(i0 + _NBUF + k, k)

            return 0

        lax.fori_loop(0, n_groups, group, 0)

    def run(inp, weights):
        pos = positions(inp)
        flat = gather(pos.reshape(b_total), weights)
        return flat.reshape(bsz, seq, dim)

    return run


@jax.jit
def kernel(input, weights):
    bsz, seq = input.shape
    vocab, dim = weights.shape
    run = _build(bsz, seq, vocab, dim)
    return run(input.astype(jnp.int32), weights.astype(jnp.float32))
